# Initial kernel scaffold; baseline (speedup 1.0000x reference)
import jax
import jax.numpy as jnp
from jax.experimental import pallas as pl


def kernel(embeddings, labels):
    raise NotImplementedError("write your pallas kernel here")


# speedup vs baseline: 5.7397x; 5.7397x over previous
import functools

import jax
import jax.numpy as jnp
from jax import lax
from jax.experimental import pallas as pl
from jax.experimental.pallas import tpu as pltpu
from jax.experimental.pallas import tpu_sc as plsc

N = 16384
D = 128
C_PAD = 112
NC, NS = 2, 16
NW = NC * NS
ROWS_W = N // NW
CHUNK = 128
NCHUNK = ROWS_W // CHUNK
ZROWS = C_PAD // NS


def _sc_segment_sums(emb, lab2d):
    mesh = plsc.VectorSubcoreMesh(core_axis_name="c", subcore_axis_name="s")

    @functools.partial(
        pl.kernel,
        mesh=mesh,
        out_type=[
            jax.ShapeDtypeStruct((NC, C_PAD, D), jnp.float32),
            jax.ShapeDtypeStruct((NC, C_PAD, D), jnp.float32),
        ],
        scratch_types=[
            pltpu.VMEM((CHUNK, D), jnp.float32),
            pltpu.VMEM((NCHUNK, CHUNK), jnp.int32),
            pltpu.VMEM((CHUNK, D), jnp.float32),
            pltpu.VMEM((ZROWS, D), jnp.float32),
            pltpu.VMEM_SHARED((C_PAD, D), jnp.float32),
            pltpu.VMEM_SHARED((C_PAD, D), jnp.float32),
        ],
    )
    def seg(emb_hbm, lab_hbm, sums_out, cnt_out,
            rows_v, idx_v, ones_v, zer_v, ssum, scnt):
        cid = lax.axis_index("c")
        sid = lax.axis_index("s")
        w = cid * NS + sid

        def fill_ones(i, carry):
            r = i // 8
            q = i % 8
            ones_v[r, pl.ds(q * 16, 16)] = jnp.ones((16,), jnp.float32)
            return carry

        lax.fori_loop(0, CHUNK * 8, fill_ones, 0)

        def fill_zeros(i, carry):
            r = i // 8
            q = i % 8
            zer_v[r, pl.ds(q * 16, 16)] = jnp.zeros((16,), jnp.float32)
            return carry

        lax.fori_loop(0, ZROWS * 8, fill_zeros, 0)

        pltpu.sync_copy(zer_v, ssum.at[pl.ds(sid * ZROWS, ZROWS)])
        pltpu.sync_copy(zer_v, scnt.at[pl.ds(sid * ZROWS, ZROWS)])
        plsc.subcore_barrier()

        pltpu.sync_copy(lab_hbm.at[pl.ds(w * NCHUNK, NCHUNK)], idx_v)

        for j in range(NCHUNK):
            pltpu.sync_copy(emb_hbm.at[pl.ds(w * ROWS_W + j * CHUNK, CHUNK)],
                            rows_v)
            pltpu.sync_copy(rows_v, ssum.at[idx_v.at[j]], add=True)
            pltpu.sync_copy(ones_v, scnt.at[idx_v.at[j]], add=True)

        plsc.subcore_barrier()

        @pl.when(sid == 0)
        def _():
            pltpu.sync_copy(ssum, sums_out.at[cid])
            pltpu.sync_copy(scnt, cnt_out.at[cid])

    return seg(emb, lab2d)


def _tc_finalize(sums2, cnt2):
    def body(s_ref, c_ref, o_ref):
        s = s_ref[...]
        c = c_ref[...]
        sums = s[0] + s[1]
        counts = (c[0] + c[1])[:, 0:1]
        safe = jnp.maximum(counts, 1.0)
        means = sums / safe
        norms = jnp.sqrt(jnp.sum(means * means, axis=1, keepdims=True))
        dot = jnp.sum(sums * means, axis=1, keepdims=True)
        dotn = dot / jnp.maximum(norms, 1e-12)
        pcm = (counts - dotn) / safe
        valid = counts >= 2.0
        nv = jnp.sum(valid.astype(jnp.float32))
        loss = jnp.sum(jnp.where(valid, pcm, jnp.zeros_like(pcm)))
        o_ref[0, 0] = jnp.where(nv > 0, loss / jnp.maximum(nv, 1.0), 0.0)

    out = pl.pallas_call(
        body,
        out_shape=jax.ShapeDtypeStruct((1, 1), jnp.float32),
        out_specs=pl.BlockSpec(memory_space=pltpu.SMEM),
    )(sums2, cnt2)
    return out[0, 0]


def kernel(embeddings, labels):
    emb = jnp.asarray(embeddings, jnp.float32)
    lab2d = jnp.asarray(labels, jnp.int32).reshape(N // CHUNK, CHUNK)
    sums2, cnt2 = _sc_segment_sums(emb, lab2d)
    return _tc_finalize(sums2, cnt2)


# speedup vs baseline: 7.5837x; 1.3213x over previous
import functools

import jax
import jax.numpy as jnp
from jax import lax
from jax.experimental import pallas as pl
from jax.experimental.pallas import tpu as pltpu
from jax.experimental.pallas import tpu_sc as plsc

N = 16384
D = 128
C_PAD = 112
NC, NS = 2, 16
NW = NC * NS
ROWS_W = N // NW
CHUNK = 128
NCHUNK = ROWS_W // CHUNK
ZROWS = C_PAD // NS


def _sc_segment_sums(emb, lab2d):
    mesh = plsc.VectorSubcoreMesh(core_axis_name="c", subcore_axis_name="s")

    @functools.partial(
        pl.kernel,
        mesh=mesh,
        out_type=jax.ShapeDtypeStruct((NC, C_PAD, D), jnp.float32),
        scratch_types=[
            pltpu.VMEM((2, CHUNK, D), jnp.float32),
            pltpu.VMEM((NCHUNK, CHUNK), jnp.int32),
            pltpu.VMEM((ZROWS, D), jnp.float32),
            pltpu.VMEM_SHARED((C_PAD, D), jnp.float32),
            pltpu.SemaphoreType.DMA,
            pltpu.SemaphoreType.DMA,
        ],
    )
    def seg(emb_hbm, lab_hbm, sums_out, rows_v, idx_v, zer_v, ssum, sem0, sem1):
        cid = lax.axis_index("c")
        sid = lax.axis_index("s")
        w = cid * NS + sid

        sems = [sem0, sem1]
        hs = [
            pltpu.async_copy(
                emb_hbm.at[pl.ds(w * ROWS_W + j * CHUNK, CHUNK)],
                rows_v.at[j], sems[j])
            for j in range(2)
        ]

        pltpu.sync_copy(lab_hbm.at[pl.ds(w * NCHUNK, NCHUNK)], idx_v)

        def fill_zeros(i, carry):
            r = i // 8
            q = i % 8
            zer_v[r, pl.ds(q * 16, 16)] = jnp.zeros((16,), jnp.float32)
            return carry

        lax.fori_loop(0, ZROWS * 8, fill_zeros, 0)

        pltpu.sync_copy(zer_v, ssum.at[pl.ds(sid * ZROWS, ZROWS)])
        plsc.subcore_barrier()

        for j in range(NCHUNK):
            b = j % 2
            hs[b].wait()
            pltpu.sync_copy(rows_v.at[b], ssum.at[idx_v.at[j]], add=True)
            if j + 2 < NCHUNK:
                hs[b] = pltpu.async_copy(
                    emb_hbm.at[pl.ds(w * ROWS_W + (j + 2) * CHUNK, CHUNK)],
                    rows_v.at[b], sems[b])

        plsc.subcore_barrier()

        @pl.when(sid == 0)
        def _():
            pltpu.sync_copy(ssum, sums_out.at[cid])

    return seg(emb, lab2d)


def _tc_finalize(sums2, lab2d):
    def body(s_ref, l_ref, o_ref):
        s = s_ref[...]
        sums = s[0] + s[1]

        iota_c = lax.broadcasted_iota(jnp.int32, (C_PAD, CHUNK), 0)

        def step(r, acc):
            lr = l_ref[pl.ds(r, 1), :]
            return acc + jnp.where(lr == iota_c, 1.0, 0.0)

        cm = lax.fori_loop(0, N // CHUNK, step,
                           jnp.zeros((C_PAD, CHUNK), jnp.float32))
        counts = jnp.sum(cm, axis=1, keepdims=True)

        safe = jnp.maximum(counts, 1.0)
        means = sums / safe
        norms = jnp.sqrt(jnp.sum(means * means, axis=1, keepdims=True))
        dot = jnp.sum(sums * means, axis=1, keepdims=True)
        dotn = dot / jnp.maximum(norms, 1e-12)
        pcm = (counts - dotn) / safe
        valid = counts >= 2.0
        nv = jnp.sum(valid.astype(jnp.float32))
        loss = jnp.sum(jnp.where(valid, pcm, jnp.zeros_like(pcm)))
        o_ref[0, 0] = jnp.where(nv > 0, loss / jnp.maximum(nv, 1.0), 0.0)

    out = pl.pallas_call(
        body,
        out_shape=jax.ShapeDtypeStruct((1, 1), jnp.float32),
        out_specs=pl.BlockSpec(memory_space=pltpu.SMEM),
    )(sums2, lab2d)
    return out[0, 0]


def kernel(embeddings, labels):
    emb = jnp.asarray(embeddings, jnp.float32)
    lab2d = jnp.asarray(labels, jnp.int32).reshape(N // CHUNK, CHUNK)
    sums2 = _sc_segment_sums(emb, lab2d)
    return _tc_finalize(sums2, lab2d)


# speedup vs baseline: 7.8795x; 1.0390x over previous
import functools

import jax
import jax.numpy as jnp
from jax import lax
from jax.experimental import pallas as pl
from jax.experimental.pallas import tpu as pltpu
from jax.experimental.pallas import tpu_sc as plsc

N = 16384
D = 128
C_PAD = 112
NC, NS = 2, 16
NW = NC * NS
ROWS_W = N // NW
CHUNK = 128
NCHUNK = ROWS_W // CHUNK
ZROWS = C_PAD // NS


def _sc_segment_sums(emb, lab2d):
    mesh = plsc.VectorSubcoreMesh(core_axis_name="c", subcore_axis_name="s")

    @functools.partial(
        pl.kernel,
        mesh=mesh,
        out_type=jax.ShapeDtypeStruct((NC, C_PAD, D), jnp.float32),
        scratch_types=[
            pltpu.VMEM((2, CHUNK, D), jnp.float32),
            pltpu.VMEM((NCHUNK, CHUNK), jnp.int32),
            pltpu.VMEM((ZROWS, D), jnp.float32),
            pltpu.VMEM_SHARED((C_PAD, D), jnp.float32),
            pltpu.SemaphoreType.DMA,
            pltpu.SemaphoreType.DMA,
        ],
    )
    def seg(emb_hbm, lab_hbm, sums_out, rows_v, idx_v, zer_v, ssum, sem0, sem1):
        cid = lax.axis_index("c")
        sid = lax.axis_index("s")
        w = cid * NS + sid

        sems = [sem0, sem1]
        hs = [
            pltpu.async_copy(
                emb_hbm.at[pl.ds(w * ROWS_W + j * CHUNK, CHUNK)],
                rows_v.at[j], sems[j])
            for j in range(2)
        ]

        pltpu.sync_copy(lab_hbm.at[pl.ds(w * NCHUNK, NCHUNK)], idx_v)

        def fill_zeros(i, carry):
            r = i // 8
            q = i % 8
            zer_v[r, pl.ds(q * 16, 16)] = jnp.zeros((16,), jnp.float32)
            return carry

        lax.fori_loop(0, ZROWS * 8, fill_zeros, 0)

        pltpu.sync_copy(zer_v, ssum.at[pl.ds(sid * ZROWS, ZROWS)])
        plsc.subcore_barrier()

        for j in range(NCHUNK):
            b = j % 2
            hs[b].wait()
            pltpu.sync_copy(rows_v.at[b], ssum.at[idx_v.at[j]], add=True)
            if j + 2 < NCHUNK:
                hs[b] = pltpu.async_copy(
                    emb_hbm.at[pl.ds(w * ROWS_W + (j + 2) * CHUNK, CHUNK)],
                    rows_v.at[b], sems[b])

        plsc.subcore_barrier()

        @pl.when(sid == 0)
        def _():
            pltpu.sync_copy(ssum, sums_out.at[cid])

    return seg(emb, lab2d)


def _tc_histogram(lab2d):
    def body(l_ref, o_ref):
        iota_c = lax.broadcasted_iota(jnp.int32, (C_PAD, CHUNK), 0)

        def step(r, acc):
            lr = l_ref[pl.ds(r, 1), :]
            return acc + jnp.where(lr == iota_c, 1.0, 0.0)

        o_ref[...] = lax.fori_loop(0, N // CHUNK, step,
                                   jnp.zeros((C_PAD, CHUNK), jnp.float32))

    return pl.pallas_call(
        body,
        out_shape=jax.ShapeDtypeStruct((C_PAD, CHUNK), jnp.float32),
    )(lab2d)


def _tc_finalize(sums2, cmat):
    def body(s_ref, c_ref, o_ref):
        s = s_ref[...]
        sums = s[0] + s[1]
        counts = jnp.sum(c_ref[...], axis=1, keepdims=True)

        safe = jnp.maximum(counts, 1.0)
        means = sums / safe
        norms = jnp.sqrt(jnp.sum(means * means, axis=1, keepdims=True))
        dot = jnp.sum(sums * means, axis=1, keepdims=True)
        dotn = dot / jnp.maximum(norms, 1e-12)
        pcm = (counts - dotn) / safe
        valid = counts >= 2.0
        nv = jnp.sum(valid.astype(jnp.float32))
        loss = jnp.sum(jnp.where(valid, pcm, jnp.zeros_like(pcm)))
        o_ref[0, 0] = jnp.where(nv > 0, loss / jnp.maximum(nv, 1.0), 0.0)

    out = pl.pallas_call(
        body,
        out_shape=jax.ShapeDtypeStruct((1, 1), jnp.float32),
        out_specs=pl.BlockSpec(memory_space=pltpu.SMEM),
    )(sums2, cmat)
    return out[0, 0]


def kernel(embeddings, labels):
    emb = jnp.asarray(embeddings, jnp.float32)
    lab2d = jnp.asarray(labels, jnp.int32).reshape(N // CHUNK, CHUNK)
    cmat = _tc_histogram(lab2d)
    sums2 = _sc_segment_sums(emb, lab2d)
    return _tc_finalize(sums2, cmat)
